# Initial kernel scaffold; baseline (speedup 1.0000x reference)
import jax
import jax.numpy as jnp
from jax.experimental import pallas as pl


def kernel(feature_map, embedding):
    raise NotImplementedError("write your pallas kernel here")


# speedup vs baseline: 3.6317x; 3.6317x over previous
import functools

import jax
import jax.numpy as jnp
from jax import lax
from jax.experimental import pallas as pl
from jax.experimental.pallas import tpu as pltpu

_MAX_SIZE = 32


def _clipped_onehot(n_rows, n_idx, base, center):
    row = lax.broadcasted_iota(jnp.int32, (n_rows, n_idx), 0)
    col = lax.broadcasted_iota(jnp.int32, (n_rows, n_idx), 1)
    idx = jnp.clip(base + row - center, -_MAX_SIZE, _MAX_SIZE) + _MAX_SIZE
    return (idx == col).astype(jnp.float32)


def _pos_kernel(emb_ref, out_ref, *, th, h, w, c):
    i = pl.program_id(1)
    n = emb_ref.shape[0]
    emb = emb_ref[...]
    oh_w = _clipped_onehot(w, n, 0, w // 2)
    rows_w = jnp.dot(oh_w, emb, preferred_element_type=jnp.float32)
    oh_h = _clipped_onehot(th, n, i * th, h // 2)
    rows_h = jnp.dot(oh_h, emb, preferred_element_type=jnp.float32)
    for j in range(th):
        out_ref[0, pl.ds(j * w, w), :] = rows_h[j : j + 1, :] + rows_w


def kernel(feature_map, embedding):
    B, C, H, W = feature_map.shape
    TH = 28
    grid = (B, H // TH)
    out = pl.pallas_call(
        functools.partial(_pos_kernel, th=TH, h=H, w=W, c=C),
        grid=grid,
        in_specs=[
            pl.BlockSpec((embedding.shape[0], C), lambda b, i: (0, 0)),
        ],
        out_specs=pl.BlockSpec((1, TH * W, C), lambda b, i: (b, i, 0)),
        out_shape=jax.ShapeDtypeStruct((B, H * W, C), jnp.float32),
        compiler_params=pltpu.CompilerParams(
            dimension_semantics=("parallel", "parallel"),
        ),
    )(embedding)
    return out


# speedup vs baseline: 3.6768x; 1.0124x over previous
import functools

import jax
import jax.numpy as jnp
from jax import lax
from jax.experimental import pallas as pl
from jax.experimental.pallas import tpu as pltpu

_MAX_SIZE = 32


def _clipped_onehot(n_rows, n_idx, base, center):
    row = lax.broadcasted_iota(jnp.int32, (n_rows, n_idx), 0)
    col = lax.broadcasted_iota(jnp.int32, (n_rows, n_idx), 1)
    idx = jnp.clip(base + row - center, -_MAX_SIZE, _MAX_SIZE) + _MAX_SIZE
    return (idx == col).astype(jnp.float32)


def _pos_kernel(emb_ref, out_ref, *, th, h, w, c):
    i = pl.program_id(1)
    n = emb_ref.shape[0]
    emb = emb_ref[...]
    oh_w = _clipped_onehot(w, n, 0, w // 2)
    rows_w = jnp.dot(oh_w, emb, preferred_element_type=jnp.float32)
    oh_h = _clipped_onehot(th, n, i * th, h // 2)
    rows_h = jnp.dot(oh_h, emb, preferred_element_type=jnp.float32)
    for j in range(th):
        out_ref[0, pl.ds(j * w, w), :] = rows_h[j : j + 1, :] + rows_w


def kernel(feature_map, embedding):
    B, C, H, W = feature_map.shape
    TH = 56
    grid = (B, H // TH)
    out = pl.pallas_call(
        functools.partial(_pos_kernel, th=TH, h=H, w=W, c=C),
        grid=grid,
        in_specs=[
            pl.BlockSpec((embedding.shape[0], C), lambda b, i: (0, 0)),
        ],
        out_specs=pl.BlockSpec((1, TH * W, C), lambda b, i: (b, i, 0)),
        out_shape=jax.ShapeDtypeStruct((B, H * W, C), jnp.float32),
        compiler_params=pltpu.CompilerParams(
            dimension_semantics=("parallel", "parallel"),
        ),
    )(embedding)
    return out


# speedup vs baseline: 3.6769x; 1.0000x over previous
import functools

import jax
import jax.numpy as jnp
from jax import lax
from jax.experimental import pallas as pl
from jax.experimental.pallas import tpu as pltpu

_MAX_SIZE = 32


def _clipped_onehot(n_rows, n_idx, base, center):
    row = lax.broadcasted_iota(jnp.int32, (n_rows, n_idx), 0)
    col = lax.broadcasted_iota(jnp.int32, (n_rows, n_idx), 1)
    idx = jnp.clip(base + row - center, -_MAX_SIZE, _MAX_SIZE) + _MAX_SIZE
    return (idx == col).astype(jnp.float32)


def _pos_kernel(emb_ref, out_ref, scratch, sems, *, th, h, w, c, nbuf, steps_per_b):
    k = pl.program_id(0)
    ch = th * w
    s = lax.rem(k, nbuf)
    b = lax.div(k, steps_per_b)
    hb = lax.rem(k, steps_per_b)
    dst = out_ref.at[b, pl.ds(hb * ch, ch), :]
    copy = pltpu.make_async_copy(scratch.at[s], dst, sems.at[s])

    @pl.when(k >= nbuf)
    def _():
        copy.wait()

    n = emb_ref.shape[0]
    emb = emb_ref[...]
    oh_w = _clipped_onehot(w, n, 0, w // 2)
    rows_w = jnp.dot(oh_w, emb, preferred_element_type=jnp.float32)
    oh_h = _clipped_onehot(th, n, hb * th, h // 2)
    rows_h = jnp.dot(oh_h, emb, preferred_element_type=jnp.float32)
    for j in range(th):
        scratch[s, pl.ds(j * w, w), :] = rows_h[j : j + 1, :] + rows_w
    copy.start()

    @pl.when(k == pl.num_programs(0) - 1)
    def _():
        for t in range(nbuf):
            pltpu.make_async_copy(scratch.at[t], dst, sems.at[t]).wait()


def kernel(feature_map, embedding):
    B, C, H, W = feature_map.shape
    TH = 28
    NBUF = 6
    steps_per_b = H // TH
    grid = (B * steps_per_b,)
    out = pl.pallas_call(
        functools.partial(
            _pos_kernel, th=TH, h=H, w=W, c=C, nbuf=NBUF, steps_per_b=steps_per_b
        ),
        grid=grid,
        in_specs=[
            pl.BlockSpec((embedding.shape[0], C), lambda k: (0, 0)),
        ],
        out_specs=pl.BlockSpec(memory_space=pltpu.MemorySpace.HBM),
        out_shape=jax.ShapeDtypeStruct((B, H * W, C), jnp.float32),
        scratch_shapes=[
            pltpu.VMEM((NBUF, TH * W, C), jnp.float32),
            pltpu.SemaphoreType.DMA((NBUF,)),
        ],
        compiler_params=pltpu.CompilerParams(
            dimension_semantics=("arbitrary",),
        ),
    )(embedding)
    return out
